# baseline (speedup 1.0000x reference)
import functools

import jax
import jax.numpy as jnp
from jax import lax
from jax.experimental import pallas as pl
from jax.experimental.pallas import tpu as pltpu
from jax.experimental.pallas import tpu_sc as plsc

_NC = 2
_NS = 16
_NW = _NC * _NS


_NSPLIT = 4
_L = 16


@functools.partial(jax.jit, static_argnames=("bpw", "chunk"))
def _gather_call(idx, *mats, bpw, chunk):
    B = idx.shape[0]
    D = mats[0].shape[1]
    vs = mats[0].shape[0]
    nchunk = bpw // chunk
    mesh = plsc.VectorSubcoreMesh(core_axis_name="c", subcore_axis_name="s")

    @functools.partial(
        pl.kernel,
        out_type=jax.ShapeDtypeStruct((B, D), jnp.float32),
        mesh=mesh,
        scratch_types=[
            pltpu.VMEM((bpw,), jnp.int32),
            pltpu.VMEM((2, _NSPLIT, chunk), jnp.int32),
            pltpu.VMEM((2, chunk, D), jnp.float32),
            pltpu.SemaphoreType.DMA,
            pltpu.SemaphoreType.DMA,
            pltpu.SemaphoreType.DMA,
            pltpu.SemaphoreType.DMA,
        ],
        compiler_params=pltpu.CompilerParams(use_tc_tiling_on_sc=False),
    )
    def gather_kernel(idx_hbm, *refs):
        mat_hbms = refs[:_NSPLIT]
        out_hbm = refs[_NSPLIT]
        idx_v, q_v, rows_v, sg0, sg1, ss0, ss1 = refs[_NSPLIT + 1:]
        wid = lax.axis_index("s") * _NC + lax.axis_index("c")
        base = wid * bpw
        pltpu.sync_copy(idx_hbm.at[pl.ds(base, bpw)], idx_v)

        def prep(i, b):
            def body(k, carry):
                v = idx_v[pl.ds(i * chunk + k * _L, _L)]
                for s in range(_NSPLIT):
                    lo = jnp.int32(s * vs)
                    q = v - lo
                    ok = (v >= lo) & (v < lo + jnp.int32(vs))
                    q = lax.select(ok, q, jnp.full((_L,), -1, jnp.int32))
                    q_v[b, s, pl.ds(k * _L, _L)] = q
                return carry

            lax.fori_loop(0, chunk // _L, body, 0, unroll=4)

        sg = (sg0, sg1)
        ss = (ss0, ss1)
        gathers = [None, None]
        stores = [None, None]
        for i in range(nchunk + 1):
            if i < nchunk:
                b = i % 2
                if stores[b] is not None:
                    stores[b].wait()
                    stores[b] = None
                prep(i, b)
                gs = []
                for s in range(_NSPLIT):
                    gs.append(pltpu.async_copy(
                        mat_hbms[s].at[
                            plsc.Indices(q_v.at[b, s], ignored_value=-1)
                        ],
                        rows_v.at[b],
                        sg[b],
                    ))
                gathers[b] = gs
            if i >= 1:
                j = i - 1
                bj = j % 2
                for g in gathers[bj]:
                    g.wait()
                stores[bj] = pltpu.async_copy(
                    rows_v.at[bj],
                    out_hbm.at[pl.ds(base + j * chunk, chunk)],
                    ss[bj],
                )
        for b in range(2):
            if stores[b] is not None:
                stores[b].wait()

    return gather_kernel(idx, *mats)


def _pick_chunk(bpw, d):
    budget = 430 * 1024 - bpw * 4
    best = 8
    c = 8
    while c <= bpw:
        if bpw % c == 0 and 2 * c * d * 4 <= budget:
            best = c
        c += 8
    return best


def kernel(x, mat):
    idx = x.reshape(-1)
    B = idx.shape[0]
    D = mat.shape[1]
    V = mat.shape[0]
    vs = V // _NSPLIT
    mats = tuple(mat[s * vs:(s + 1) * vs] for s in range(_NSPLIT))
    bpw = B // _NW
    chunk = _pick_chunk(bpw, D)
    return _gather_call(idx, *mats, bpw=bpw, chunk=chunk)
